# baseline (speedup 1.0000x reference)
import functools

import numpy as np
import jax
import jax.numpy as jnp
from jax import lax
from jax.experimental import pallas as pl
from jax.experimental.pallas import tpu as pltpu
from jax.experimental.pallas import tpu_sc as plsc

_TOKENS = 32768
_NUM_OPS = 64
_WIDTH = 64
_NC = 2
_NS = 16
_NW = _NC * _NS
_TPW = _TOKENS // _NW
_LANES = 16
_GROUPS = _TPW // _LANES
_WSTRIDE = 2


def _lut_body(mv_ref, mk_ref, out_ref):
    mv = mv_ref[...]
    mk = mk_ref[...]
    iota_v = lax.broadcasted_iota(jnp.int32, (_NUM_OPS, _WIDTH, _NUM_OPS), 2)
    cond = (mv[:, :, None] == iota_v) | (mk[:, :, None] != jnp.int32(0))
    iota_o = lax.broadcasted_iota(jnp.int32, (32, 1, 1), 0)
    w = jnp.left_shift(jnp.int32(1), iota_o)
    lo = jnp.sum(jnp.where(cond[:32], w, jnp.int32(0)), axis=0, dtype=jnp.int32)
    hi = jnp.sum(jnp.where(cond[32:], w, jnp.int32(0)), axis=0, dtype=jnp.int32)
    out_ref[...] = jnp.stack([lo, hi], axis=0)


def _build_lut(mv32, mk32):
    return pl.pallas_call(
        _lut_body,
        out_shape=jax.ShapeDtypeStruct((2, _WIDTH, _NUM_OPS), jnp.int32),
    )(mv32, mk32)


def _bit_pos(acc):
    is31 = acc == jnp.int32(np.int32(-(2**31)))
    f = acc.astype(jnp.float32)
    bits = plsc.bitcast(f, jnp.int32)
    e = ((bits >> jnp.int32(23)) & jnp.int32(0xFF)) - jnp.int32(127)
    return jnp.where(is31, jnp.int32(31), e)


_HTOK = 512
_HGROUPS = _HTOK // _LANES


def _sc_body(t_hbm, lut_hbm, act_hbm, out_hbm, chunk, lut_lo, lut_hi, acc_lo_b,
             acc_hi_b, outv, actv):
    wid = lax.axis_index("s") * np.int32(_NC) + lax.axis_index("c")
    base = wid * np.int32(_TPW)
    pltpu.sync_copy(lut_hbm.at[np.int32(0)], lut_lo)
    pltpu.sync_copy(lut_hbm.at[np.int32(1)], lut_hi)
    pltpu.sync_copy(act_hbm, actv)

    act_rows = [actv[pl.ds(r * _LANES, _LANES)] for r in range(_WIDTH // _LANES)]
    iota = lax.iota(jnp.int32, _LANES)

    @pl.loop(jnp.int32(0), jnp.int32(_TPW // _HTOK), step=jnp.int32(1))
    def _per_half(h):
        hbase = base + h * np.int32(_HTOK)
        pltpu.sync_copy(t_hbm.at[pl.ds(hbase, _HTOK)], chunk)

        @pl.loop(jnp.int32(0), jnp.int32(_HGROUPS), step=jnp.int32(1))
        def _init(g):
            off = g * np.int32(_LANES)
            ones = jnp.full((_LANES,), np.int32(-1), jnp.int32)
            acc_lo_b[pl.ds(off, _LANES)] = ones
            acc_hi_b[pl.ds(off, _LANES)] = ones

        for c in range(_WIDTH):
            act_c = act_rows[c // _LANES][c % _LANES]

            @pl.when(act_c != jnp.int32(0))
            def _do_channel(c=c):
                col = jnp.full((_LANES,), np.int32(c * _WSTRIDE), jnp.int32)

                @pl.loop(jnp.int32(0), jnp.int32(_HGROUPS), step=jnp.int32(1))
                def _per_group(g):
                    off = g * np.int32(_LANES)
                    rows = iota + off
                    v = plsc.load_gather(chunk, [rows, col])
                    lidx = v + np.int32(c * _NUM_OPS)
                    acc_lo_b[pl.ds(off, _LANES)] = (
                        acc_lo_b[pl.ds(off, _LANES)]
                        & plsc.load_gather(lut_lo, [lidx])
                    )
                    acc_hi_b[pl.ds(off, _LANES)] = (
                        acc_hi_b[pl.ds(off, _LANES)]
                        & plsc.load_gather(lut_hi, [lidx])
                    )

        @pl.loop(jnp.int32(0), jnp.int32(_HGROUPS), step=jnp.int32(1))
        def _extract(g):
            off = g * np.int32(_LANES)
            a_lo = acc_lo_b[pl.ds(off, _LANES)]
            a_hi = acc_hi_b[pl.ds(off, _LANES)]
            outv[pl.ds(off, _LANES)] = jnp.where(
                a_lo != jnp.int32(0), _bit_pos(a_lo), jnp.int32(32) + _bit_pos(a_hi)
            )

        pltpu.sync_copy(outv, out_hbm.at[pl.ds(hbase, _HTOK)])


@functools.partial(
    pl.kernel,
    out_type=jax.ShapeDtypeStruct((_TOKENS,), jnp.int32),
    mesh=plsc.VectorSubcoreMesh(core_axis_name="c", subcore_axis_name="s"),
    scratch_types=[
        pltpu.VMEM((_HTOK, _WIDTH * _WSTRIDE), jnp.int32),
        pltpu.VMEM((_WIDTH * _NUM_OPS,), jnp.int32),
        pltpu.VMEM((_WIDTH * _NUM_OPS,), jnp.int32),
        pltpu.VMEM((_HTOK,), jnp.int32),
        pltpu.VMEM((_HTOK,), jnp.int32),
        pltpu.VMEM((_HTOK,), jnp.int32),
        pltpu.VMEM((_WIDTH,), jnp.int32),
    ],
    compiler_params=pltpu.CompilerParams(needs_layout_passes=False),
)
def _sc_match(t_hbm, lut_hbm, act_hbm, out_hbm, *scratch):
    _sc_body(t_hbm, lut_hbm, act_hbm, out_hbm, *scratch)


def kernel(tensor, match_values, channel_masks):
    t2 = lax.bitcast_convert_type(tensor, jnp.int32).reshape(_TOKENS, _WIDTH * 2)
    mv32 = match_values.astype(jnp.int32)
    mk32 = channel_masks.astype(jnp.int32)
    lut = _build_lut(mv32, mk32)
    act32 = jnp.any(~channel_masks, axis=0).astype(jnp.int32)
    out = _sc_match(t2, lut.reshape(2, _WIDTH * _NUM_OPS), act32)
    return out.astype(tensor.dtype)


# speedup vs baseline: 2.0917x; 2.0917x over previous
import functools

import numpy as np
import jax
import jax.numpy as jnp
from jax import lax
from jax.experimental import pallas as pl
from jax.experimental.pallas import tpu as pltpu
from jax.experimental.pallas import tpu_sc as plsc

_TOKENS = 32768
_NUM_OPS = 64
_WIDTH = 64
_NC = 2
_NS = 16
_NW = _NC * _NS
_TPW = _TOKENS // _NW
_LANES = 16
_GROUPS = _TPW // _LANES
_WSTRIDE = 2


def _lut_body(mv_ref, mk_ref, out_ref, act_ref):
    mv = mv_ref[...]
    mk = mk_ref[...]
    iota_v = lax.broadcasted_iota(jnp.int32, (_NUM_OPS, _WIDTH, _NUM_OPS), 2)
    cond = (mv[:, :, None] == iota_v) | (mk[:, :, None] != jnp.int32(0))
    iota_o = lax.broadcasted_iota(jnp.int32, (32, 1, 1), 0)
    w = jnp.left_shift(jnp.int32(1), iota_o)
    lo = jnp.sum(jnp.where(cond[:32], w, jnp.int32(0)), axis=0, dtype=jnp.int32)
    hi = jnp.sum(jnp.where(cond[32:], w, jnp.int32(0)), axis=0, dtype=jnp.int32)
    out_ref[...] = jnp.stack([lo, hi], axis=0)
    act_ref[...] = jnp.where(
        jnp.min(mk, axis=0) == jnp.int32(0), jnp.int32(1), jnp.int32(0)
    )


def _build_lut(mv32, mk32):
    return pl.pallas_call(
        _lut_body,
        out_shape=(
            jax.ShapeDtypeStruct((2, _WIDTH, _NUM_OPS), jnp.int32),
            jax.ShapeDtypeStruct((_WIDTH,), jnp.int32),
        ),
    )(mv32, mk32)


def _bit_pos(acc):
    is31 = acc == jnp.int32(np.int32(-(2**31)))
    f = acc.astype(jnp.float32)
    bits = plsc.bitcast(f, jnp.int32)
    e = ((bits >> jnp.int32(23)) & jnp.int32(0xFF)) - jnp.int32(127)
    return jnp.where(is31, jnp.int32(31), e)


_HTOK = 512
_HGROUPS = _HTOK // _LANES


def _sc_body(t_hbm, lut_hbm, act_hbm, out_hbm, chunk, lut_lo, lut_hi, acc_lo_b,
             acc_hi_b, outv, actv):
    wid = lax.axis_index("s") * np.int32(_NC) + lax.axis_index("c")
    base = wid * np.int32(_TPW)
    pltpu.sync_copy(lut_hbm.at[np.int32(0)], lut_lo)
    pltpu.sync_copy(lut_hbm.at[np.int32(1)], lut_hi)
    pltpu.sync_copy(act_hbm, actv)
    pltpu.sync_copy(t_hbm.at[pl.ds(base * np.int32(_WIDTH), _TPW * _WIDTH)], chunk)

    act_rows = [actv[pl.ds(r * _LANES, _LANES)] for r in range(_WIDTH // _LANES)]
    iota_w = lax.iota(jnp.int32, _LANES) * np.int32(_WIDTH)

    @pl.loop(jnp.int32(0), jnp.int32(_GROUPS), step=jnp.int32(1))
    def _init(g):
        off = g * np.int32(_LANES)
        ones = jnp.full((_LANES,), np.int32(-1), jnp.int32)
        acc_lo_b[pl.ds(off, _LANES)] = ones
        acc_hi_b[pl.ds(off, _LANES)] = ones

    for c in range(_WIDTH):
        act_c = act_rows[c // _LANES][c % _LANES]

        @pl.when(act_c != jnp.int32(0))
        def _do_channel(c=c):
            @pl.loop(jnp.int32(0), jnp.int32(_GROUPS), step=jnp.int32(1))
            def _per_group(g):
                off = g * np.int32(_LANES)
                tidx = iota_w + (off * np.int32(_WIDTH) + np.int32(c))
                v = plsc.load_gather(chunk, [tidx])
                lidx = v + np.int32(c * _NUM_OPS)
                acc_lo_b[pl.ds(off, _LANES)] = (
                    acc_lo_b[pl.ds(off, _LANES)]
                    & plsc.load_gather(lut_lo, [lidx])
                )
                acc_hi_b[pl.ds(off, _LANES)] = (
                    acc_hi_b[pl.ds(off, _LANES)]
                    & plsc.load_gather(lut_hi, [lidx])
                )

    @pl.loop(jnp.int32(0), jnp.int32(_GROUPS), step=jnp.int32(1))
    def _extract(g):
        off = g * np.int32(_LANES)
        a_lo = acc_lo_b[pl.ds(off, _LANES)]
        a_hi = acc_hi_b[pl.ds(off, _LANES)]
        outv[pl.ds(off, _LANES)] = jnp.where(
            a_lo != jnp.int32(0), _bit_pos(a_lo), jnp.int32(32) + _bit_pos(a_hi)
        )

    pltpu.sync_copy(outv, out_hbm.at[pl.ds(base, _TPW)])


@functools.partial(
    pl.kernel,
    out_type=jax.ShapeDtypeStruct((_TOKENS,), jnp.int32),
    mesh=plsc.VectorSubcoreMesh(core_axis_name="c", subcore_axis_name="s"),
    scratch_types=[
        pltpu.VMEM((_TPW * _WIDTH,), jnp.int32),
        pltpu.VMEM((_WIDTH * _NUM_OPS,), jnp.int32),
        pltpu.VMEM((_WIDTH * _NUM_OPS,), jnp.int32),
        pltpu.VMEM((_TPW,), jnp.int32),
        pltpu.VMEM((_TPW,), jnp.int32),
        pltpu.VMEM((_TPW,), jnp.int32),
        pltpu.VMEM((_WIDTH,), jnp.int32),
    ],
    compiler_params=pltpu.CompilerParams(needs_layout_passes=False),
)
def _sc_match(t_hbm, lut_hbm, act_hbm, out_hbm, *scratch):
    _sc_body(t_hbm, lut_hbm, act_hbm, out_hbm, *scratch)


def kernel(tensor, match_values, channel_masks):
    t32 = tensor.astype(jnp.int32)
    mv32 = match_values.astype(jnp.int32)
    mk32 = channel_masks.astype(jnp.int32)
    lut, act32 = _build_lut(mv32, mk32)
    out = _sc_match(t32.reshape(-1), lut.reshape(2, _WIDTH * _NUM_OPS), act32)
    return out.astype(tensor.dtype)
